# Initial kernel scaffold; baseline (speedup 1.0000x reference)
import functools

import jax
import jax.numpy as jnp
from jax import lax
from jax.experimental import pallas as pl
from jax.experimental.pallas import tpu as pltpu
from jax.experimental.pallas import tpu_sc as plsc

N = 10000
E = 320000
D = 128
NC = 2
NS = 16
NW = NC * NS
EW = E // NW
K = 80
CH = EW // K
RT = N // NS

_mesh = plsc.VectorSubcoreMesh(core_axis_name="c", subcore_axis_name="s")


@functools.partial(
    pl.kernel,
    out_type=jax.ShapeDtypeStruct((NC, N, 16), jnp.float32),
    mesh=_mesh,
    scratch_types=[
        pltpu.VMEM((CH, K), jnp.int32),
        pltpu.VMEM((K, 16), jnp.float32),
        pltpu.VMEM_SHARED((N, 16), jnp.float32),
    ],
)
def _sc_hist(dst_hbm, ones_hbm, zeros_hbm, out_hbm, idx_v, row_v, hist_sh):
    c = lax.axis_index("c")
    s = lax.axis_index("s")
    wid = s * NC + c
    pltpu.sync_copy(zeros_hbm, row_v)
    base = s * RT
    for k in range(RT // K):
        pltpu.sync_copy(row_v, hist_sh.at[pl.ds(base + k * K, K)])
    rem = RT % K
    if rem:
        pltpu.sync_copy(row_v.at[pl.ds(0, rem)],
                        hist_sh.at[pl.ds(base + (RT // K) * K, rem)])
    pltpu.sync_copy(dst_hbm.at[wid], idx_v)
    pltpu.sync_copy(ones_hbm, row_v)
    plsc.subcore_barrier()

    def chunk(ci, carry):
        pltpu.sync_copy(row_v, hist_sh.at[idx_v.at[ci]], add=True)
        return carry

    lax.fori_loop(0, CH, chunk, 0)
    plsc.subcore_barrier()
    pltpu.sync_copy(hist_sh.at[pl.ds(base, RT)], out_hbm.at[c, pl.ds(base, RT)])


@functools.partial(
    pl.kernel,
    out_type=jax.ShapeDtypeStruct((NC, N, D), jnp.float32),
    mesh=_mesh,
    scratch_types=[
        pltpu.VMEM((CH, K), jnp.int32),
        pltpu.VMEM((CH, K), jnp.int32),
        pltpu.VMEM((K, D), jnp.float32),
        pltpu.VMEM_SHARED((N, D), jnp.float32),
    ],
)
def _sc_agg(y_hbm, src_hbm, dst_hbm, zeros_hbm, out_hbm,
            src_v, dst_v, row_v, acc_sh):
    c = lax.axis_index("c")
    s = lax.axis_index("s")
    wid = s * NC + c
    pltpu.sync_copy(zeros_hbm, row_v)
    base = s * RT
    for k in range(RT // K):
        pltpu.sync_copy(row_v, acc_sh.at[pl.ds(base + k * K, K)])
    rem = RT % K
    if rem:
        pltpu.sync_copy(row_v.at[pl.ds(0, rem)],
                        acc_sh.at[pl.ds(base + (RT // K) * K, rem)])
    pltpu.sync_copy(src_hbm.at[wid], src_v)
    pltpu.sync_copy(dst_hbm.at[wid], dst_v)
    plsc.subcore_barrier()

    def chunk(ci, carry):
        pltpu.sync_copy(y_hbm.at[src_v.at[ci]], row_v)
        pltpu.sync_copy(row_v, acc_sh.at[dst_v.at[ci]], add=True)
        return carry

    lax.fori_loop(0, CH, chunk, 0)
    plsc.subcore_barrier()
    pltpu.sync_copy(acc_sh.at[pl.ds(base, RT)], out_hbm.at[c, pl.ds(base, RT)])


_BLK = 1000
_GRID = N // _BLK


def _deg_r(h0, h1):
    deg = h0[:, 0:1] + h1[:, 0:1] + 1.0
    return lax.rsqrt(deg)


def _tc_a_body(h0_ref, h1_ref, x_ref, w1_ref, yp_ref):
    r = _deg_r(h0_ref[...], h1_ref[...])
    y = lax.dot_general(x_ref[...], w1_ref[...], (((1,), (1,)), ((), ())),
                        preferred_element_type=jnp.float32)
    yp_ref[...] = r * y


def _tc_b_body(p0_ref, p1_ref, yp1_ref, h0_ref, h1_ref, b1_ref, w2_ref,
               yp2_ref):
    r = _deg_r(h0_ref[...], h1_ref[...])
    s1 = r * (p0_ref[...] + p1_ref[...] + yp1_ref[...]) + b1_ref[...]
    h = jnp.maximum(s1, 0.0)
    z = lax.dot_general(h, w2_ref[...], (((1,), (1,)), ((), ())),
                        preferred_element_type=jnp.float32)
    yp2_ref[...] = r * z


def _tc_c_body(q0_ref, q1_ref, yp2_ref, h0_ref, h1_ref, b2_ref, out_ref):
    r = _deg_r(h0_ref[...], h1_ref[...])
    out_ref[...] = r * (q0_ref[...] + q1_ref[...] + yp2_ref[...]) + b2_ref[...]


def _rowspec(w):
    return pl.BlockSpec((_BLK, w), lambda i: (i, 0))


def _bcast(shape):
    return pl.BlockSpec(shape, lambda i: (0, 0))


def kernel(x, edge_index, W1, b1, W2, b2):
    src = edge_index[0].reshape(NW, CH, K)
    dst = edge_index[1].reshape(NW, CH, K)
    ones16 = jnp.ones((K, 16), jnp.float32)
    zeros16 = jnp.zeros((K, 16), jnp.float32)
    zerosD = jnp.zeros((K, D), jnp.float32)

    hist = _sc_hist(dst, ones16, zeros16)
    h0, h1 = hist[0], hist[1]

    yp1 = pl.pallas_call(
        _tc_a_body,
        grid=(_GRID,),
        in_specs=[_rowspec(16), _rowspec(16), _rowspec(D), _bcast((D, D))],
        out_specs=_rowspec(D),
        out_shape=jax.ShapeDtypeStruct((N, D), jnp.float32),
    )(h0, h1, x, W1)

    P = _sc_agg(yp1, src, dst, zerosD)

    yp2 = pl.pallas_call(
        _tc_b_body,
        grid=(_GRID,),
        in_specs=[_rowspec(D), _rowspec(D), _rowspec(D), _rowspec(16),
                  _rowspec(16), _bcast((1, D)), _bcast((D, D))],
        out_specs=_rowspec(D),
        out_shape=jax.ShapeDtypeStruct((N, D), jnp.float32),
    )(P[0], P[1], yp1, h0, h1, b1.reshape(1, D), W2)

    Q = _sc_agg(yp2, src, dst, zerosD)

    out = pl.pallas_call(
        _tc_c_body,
        grid=(_GRID,),
        in_specs=[_rowspec(D), _rowspec(D), _rowspec(D), _rowspec(16),
                  _rowspec(16), _bcast((1, D))],
        out_specs=_rowspec(D),
        out_shape=jax.ShapeDtypeStruct((N, D), jnp.float32),
    )(Q[0], Q[1], yp2, h0, h1, b2.reshape(1, D))
    return out


# speedup vs baseline: 18.4205x; 18.4205x over previous
import functools

import jax
import jax.numpy as jnp
from jax import lax
from jax.experimental import pallas as pl
from jax.experimental.pallas import tpu as pltpu
from jax.experimental.pallas import tpu_sc as plsc

N = 10000
E = 320000
D = 128
NC = 2
NS = 16
NW = NC * NS
EW = E // NW
K = 80
CH = EW // K
BT = 640
NB = BT // K

_mesh = plsc.VectorSubcoreMesh(core_axis_name="c", subcore_axis_name="s")


@functools.partial(
    pl.kernel,
    out_type=jax.ShapeDtypeStruct((NC, N, 16), jnp.float32),
    mesh=_mesh,
    scratch_types=[
        pltpu.VMEM((CH, K), jnp.int32),
        pltpu.VMEM((K, 16), jnp.float32),
        pltpu.VMEM_SHARED((N, 16), jnp.float32),
    ],
)
def _sc_hist(dst_hbm, ones_hbm, zeros_hbm, out_hbm, idx_v, row_v, hist_sh):
    c = lax.axis_index("c")
    s = lax.axis_index("s")
    wid = s * NC + c
    pltpu.sync_copy(zeros_hbm, row_v)
    base = s * BT
    for k in range(NB):
        @pl.when(base + k * K < N)
        def _():
            pltpu.sync_copy(row_v, hist_sh.at[pl.ds(base + k * K, K)])
    pltpu.sync_copy(dst_hbm.at[wid], idx_v)
    pltpu.sync_copy(ones_hbm, row_v)
    plsc.subcore_barrier()

    def chunk(ci, carry):
        pltpu.sync_copy(row_v, hist_sh.at[idx_v.at[ci]], add=True)
        return carry

    lax.fori_loop(0, CH, chunk, 0)
    plsc.subcore_barrier()
    for k in range(NB):
        @pl.when(base + k * K < N)
        def _():
            pltpu.sync_copy(hist_sh.at[pl.ds(base + k * K, K)],
                            out_hbm.at[c, pl.ds(base + k * K, K)])


@functools.partial(
    pl.kernel,
    out_type=jax.ShapeDtypeStruct((NC, N, D), jnp.float32),
    mesh=_mesh,
    scratch_types=[
        pltpu.VMEM((CH, K), jnp.int32),
        pltpu.VMEM((CH, K), jnp.int32),
        pltpu.VMEM((K, D), jnp.float32),
        pltpu.VMEM_SHARED((N, D), jnp.float32),
    ],
)
def _sc_agg(y_hbm, src_hbm, dst_hbm, zeros_hbm, out_hbm,
            src_v, dst_v, row_v, acc_sh):
    c = lax.axis_index("c")
    s = lax.axis_index("s")
    wid = s * NC + c
    pltpu.sync_copy(zeros_hbm, row_v)
    base = s * BT
    for k in range(NB):
        @pl.when(base + k * K < N)
        def _():
            pltpu.sync_copy(row_v, acc_sh.at[pl.ds(base + k * K, K)])
    pltpu.sync_copy(src_hbm.at[wid], src_v)
    pltpu.sync_copy(dst_hbm.at[wid], dst_v)
    plsc.subcore_barrier()

    def chunk(ci, carry):
        pltpu.sync_copy(y_hbm.at[src_v.at[ci]], row_v)
        pltpu.sync_copy(row_v, acc_sh.at[dst_v.at[ci]], add=True)
        return carry

    lax.fori_loop(0, CH, chunk, 0)
    plsc.subcore_barrier()
    for k in range(NB):
        @pl.when(base + k * K < N)
        def _():
            pltpu.sync_copy(acc_sh.at[pl.ds(base + k * K, K)],
                            out_hbm.at[c, pl.ds(base + k * K, K)])


_BLK = 1000
_GRID = N // _BLK


def _deg_r(h0, h1):
    deg = h0[:, 0:1] + h1[:, 0:1] + 1.0
    return lax.rsqrt(deg)


def _tc_a_body(h0_ref, h1_ref, x_ref, w1_ref, yp_ref):
    r = _deg_r(h0_ref[...], h1_ref[...])
    y = lax.dot_general(x_ref[...], w1_ref[...], (((1,), (1,)), ((), ())),
                        preferred_element_type=jnp.float32)
    yp_ref[...] = r * y


def _tc_b_body(p0_ref, p1_ref, yp1_ref, h0_ref, h1_ref, b1_ref, w2_ref,
               yp2_ref):
    r = _deg_r(h0_ref[...], h1_ref[...])
    s1 = r * (p0_ref[...] + p1_ref[...] + yp1_ref[...]) + b1_ref[...]
    h = jnp.maximum(s1, 0.0)
    z = lax.dot_general(h, w2_ref[...], (((1,), (1,)), ((), ())),
                        preferred_element_type=jnp.float32)
    yp2_ref[...] = r * z


def _tc_c_body(q0_ref, q1_ref, yp2_ref, h0_ref, h1_ref, b2_ref, out_ref):
    r = _deg_r(h0_ref[...], h1_ref[...])
    out_ref[...] = r * (q0_ref[...] + q1_ref[...] + yp2_ref[...]) + b2_ref[...]


def _rowspec(w):
    return pl.BlockSpec((_BLK, w), lambda i: (i, 0))


def _bcast(shape):
    return pl.BlockSpec(shape, lambda i: (0, 0))


def kernel(x, edge_index, W1, b1, W2, b2):
    src = edge_index[0].reshape(NW, CH, K)
    dst = edge_index[1].reshape(NW, CH, K)
    ones16 = jnp.ones((K, 16), jnp.float32)
    zeros16 = jnp.zeros((K, 16), jnp.float32)
    zerosD = jnp.zeros((K, D), jnp.float32)

    hist = _sc_hist(dst, ones16, zeros16)
    h0, h1 = hist[0], hist[1]

    yp1 = pl.pallas_call(
        _tc_a_body,
        grid=(_GRID,),
        in_specs=[_rowspec(16), _rowspec(16), _rowspec(D), _bcast((D, D))],
        out_specs=_rowspec(D),
        out_shape=jax.ShapeDtypeStruct((N, D), jnp.float32),
    )(h0, h1, x, W1)

    P = _sc_agg(yp1, src, dst, zerosD)

    yp2 = pl.pallas_call(
        _tc_b_body,
        grid=(_GRID,),
        in_specs=[_rowspec(D), _rowspec(D), _rowspec(D), _rowspec(16),
                  _rowspec(16), _bcast((1, D)), _bcast((D, D))],
        out_specs=_rowspec(D),
        out_shape=jax.ShapeDtypeStruct((N, D), jnp.float32),
    )(P[0], P[1], yp1, h0, h1, b1.reshape(1, D), W2)

    Q = _sc_agg(yp2, src, dst, zerosD)

    out = pl.pallas_call(
        _tc_c_body,
        grid=(_GRID,),
        in_specs=[_rowspec(D), _rowspec(D), _rowspec(D), _rowspec(16),
                  _rowspec(16), _bcast((1, D))],
        out_specs=_rowspec(D),
        out_shape=jax.ShapeDtypeStruct((N, D), jnp.float32),
    )(Q[0], Q[1], yp2, h0, h1, b2.reshape(1, D))
    return out


# speedup vs baseline: 23.6765x; 1.2853x over previous
import functools

import jax
import jax.numpy as jnp
from jax import lax
from jax.experimental import pallas as pl
from jax.experimental.pallas import tpu as pltpu
from jax.experimental.pallas import tpu_sc as plsc

N = 10000
E = 320000
D = 128
NC = 2
NS = 16
NW = NC * NS
EW = E // NW
K = 80
CH = EW // K
BT = 640
NB = BT // K

_mesh = plsc.VectorSubcoreMesh(core_axis_name="c", subcore_axis_name="s")


@functools.partial(
    pl.kernel,
    out_type=jax.ShapeDtypeStruct((NC, N, 16), jnp.float32),
    mesh=_mesh,
    scratch_types=[
        pltpu.VMEM((CH, K), jnp.int32),
        pltpu.VMEM((K, 16), jnp.float32),
        pltpu.VMEM_SHARED((N, 16), jnp.float32),
    ],
)
def _sc_hist(dst_hbm, ones_hbm, zeros_hbm, out_hbm, idx_v, row_v, hist_sh):
    c = lax.axis_index("c")
    s = lax.axis_index("s")
    wid = s * NC + c
    pltpu.sync_copy(zeros_hbm, row_v)
    base = s * BT
    for k in range(NB):
        @pl.when(base + k * K < N)
        def _():
            pltpu.sync_copy(row_v, hist_sh.at[pl.ds(base + k * K, K)])
    pltpu.sync_copy(dst_hbm.at[wid], idx_v)
    pltpu.sync_copy(ones_hbm, row_v)
    plsc.subcore_barrier()

    def chunk(ci, carry):
        pltpu.sync_copy(row_v, hist_sh.at[idx_v.at[ci]], add=True)
        return carry

    lax.fori_loop(0, CH, chunk, 0)
    plsc.subcore_barrier()
    for k in range(NB):
        @pl.when(base + k * K < N)
        def _():
            pltpu.sync_copy(hist_sh.at[pl.ds(base + k * K, K)],
                            out_hbm.at[c, pl.ds(base + k * K, K)])


@functools.partial(
    pl.kernel,
    out_type=jax.ShapeDtypeStruct((NC, N, D), jnp.float32),
    mesh=_mesh,
    scratch_types=[
        pltpu.VMEM((2, K), jnp.int32),
        pltpu.VMEM((2, K), jnp.int32),
        pltpu.VMEM((K, D), jnp.float32),
        pltpu.VMEM((K, D), jnp.float32),
        pltpu.VMEM_SHARED((N, D), jnp.float32),
        pltpu.SemaphoreType.DMA,
        pltpu.SemaphoreType.DMA,
        pltpu.SemaphoreType.DMA,
        pltpu.SemaphoreType.DMA,
    ],
)
def _sc_agg(y_hbm, ei_hbm, zeros_hbm, out_hbm,
            i0, i1, b0, b1, acc_sh, si0, si1, sg0, sg1):
    c = lax.axis_index("c")
    s = lax.axis_index("s")
    wid = s * NC + c
    pltpu.sync_copy(zeros_hbm, b0)
    base = s * BT
    for k in range(NB):
        @pl.when(base + k * K < N)
        def _():
            pltpu.sync_copy(b0, acc_sh.at[pl.ds(base + k * K, K)])

    def fetch_idx(ci, islot, isem):
        pltpu.async_copy(ei_hbm.at[wid, ci], islot, isem)

    def wait_idx(ci, islot, isem):
        pltpu.make_async_copy(ei_hbm.at[wid, ci], islot, isem).wait()

    fetch_idx(0, i0, si0)
    fetch_idx(1, i1, si1)
    plsc.subcore_barrier()
    wait_idx(0, i0, si0)
    pltpu.async_copy(y_hbm.at[i0.at[0]], b0, sg0)

    def step(ci, islot, isem, bslot, bsem, nislot, nisem, nbslot, nbsem):
        wait_idx(ci + 1, nislot, nisem)
        pltpu.async_copy(y_hbm.at[nislot.at[0]], nbslot, nbsem)
        pltpu.make_async_copy(y_hbm.at[islot.at[0]], bslot, bsem).wait()
        pltpu.sync_copy(bslot, acc_sh.at[islot.at[1]], add=True)
        @pl.when(ci + 2 < CH)
        def _():
            fetch_idx(ci + 2, islot, isem)

    def pair(i, carry):
        ci0 = 2 * i
        step(ci0, i0, si0, b0, sg0, i1, si1, b1, sg1)
        step(ci0 + 1, i1, si1, b1, sg1, i0, si0, b0, sg0)
        return carry

    lax.fori_loop(0, (CH - 1) // 2, pair, 0)
    pltpu.make_async_copy(y_hbm.at[i0.at[0]], b0, sg0).wait()
    pltpu.sync_copy(b0, acc_sh.at[i0.at[1]], add=True)
    plsc.subcore_barrier()
    for k in range(NB):
        @pl.when(base + k * K < N)
        def _():
            pltpu.sync_copy(acc_sh.at[pl.ds(base + k * K, K)],
                            out_hbm.at[c, pl.ds(base + k * K, K)])


_BLK = 1000
_GRID = N // _BLK


def _deg_r(h0, h1):
    deg = h0[:, 0:1] + h1[:, 0:1] + 1.0
    return lax.rsqrt(deg)


def _tc_a_body(h0_ref, h1_ref, x_ref, w1_ref, yp_ref):
    r = _deg_r(h0_ref[...], h1_ref[...])
    y = lax.dot_general(x_ref[...], w1_ref[...], (((1,), (1,)), ((), ())),
                        preferred_element_type=jnp.float32)
    yp_ref[...] = r * y


def _tc_b_body(p0_ref, p1_ref, yp1_ref, h0_ref, h1_ref, b1_ref, w2_ref,
               yp2_ref):
    r = _deg_r(h0_ref[...], h1_ref[...])
    s1 = r * (p0_ref[...] + p1_ref[...] + yp1_ref[...]) + b1_ref[...]
    h = jnp.maximum(s1, 0.0)
    z = lax.dot_general(h, w2_ref[...], (((1,), (1,)), ((), ())),
                        preferred_element_type=jnp.float32)
    yp2_ref[...] = r * z


def _tc_c_body(q0_ref, q1_ref, yp2_ref, h0_ref, h1_ref, b2_ref, out_ref):
    r = _deg_r(h0_ref[...], h1_ref[...])
    out_ref[...] = r * (q0_ref[...] + q1_ref[...] + yp2_ref[...]) + b2_ref[...]


def _rowspec(w):
    return pl.BlockSpec((_BLK, w), lambda i: (i, 0))


def _bcast(shape):
    return pl.BlockSpec(shape, lambda i: (0, 0))


def kernel(x, edge_index, W1, b1, W2, b2):
    src = edge_index[0].reshape(NW, CH, K)
    dst = edge_index[1].reshape(NW, CH, K)
    ei = jnp.stack([src, dst], axis=2)
    ones16 = jnp.ones((K, 16), jnp.float32)
    zeros16 = jnp.zeros((K, 16), jnp.float32)
    zerosD = jnp.zeros((K, D), jnp.float32)

    hist = _sc_hist(dst, ones16, zeros16)
    h0, h1 = hist[0], hist[1]

    yp1 = pl.pallas_call(
        _tc_a_body,
        grid=(_GRID,),
        in_specs=[_rowspec(16), _rowspec(16), _rowspec(D), _bcast((D, D))],
        out_specs=_rowspec(D),
        out_shape=jax.ShapeDtypeStruct((N, D), jnp.float32),
    )(h0, h1, x, W1)

    P = _sc_agg(yp1, ei, zerosD)

    yp2 = pl.pallas_call(
        _tc_b_body,
        grid=(_GRID,),
        in_specs=[_rowspec(D), _rowspec(D), _rowspec(D), _rowspec(16),
                  _rowspec(16), _bcast((1, D)), _bcast((D, D))],
        out_specs=_rowspec(D),
        out_shape=jax.ShapeDtypeStruct((N, D), jnp.float32),
    )(P[0], P[1], yp1, h0, h1, b1.reshape(1, D), W2)

    Q = _sc_agg(yp2, ei, zerosD)

    out = pl.pallas_call(
        _tc_c_body,
        grid=(_GRID,),
        in_specs=[_rowspec(D), _rowspec(D), _rowspec(D), _rowspec(16),
                  _rowspec(16), _bcast((1, D))],
        out_specs=_rowspec(D),
        out_shape=jax.ShapeDtypeStruct((N, D), jnp.float32),
    )(Q[0], Q[1], yp2, h0, h1, b2.reshape(1, D))
    return out
